# Initial kernel scaffold; baseline (speedup 1.0000x reference)
import jax
import jax.numpy as jnp
from jax.experimental import pallas as pl


def kernel(users, item, user_emb, item_emb):
    raise NotImplementedError("write your pallas kernel here")


# speedup vs baseline: 1.1696x; 1.1696x over previous
import functools

import jax
import jax.numpy as jnp
from jax import lax
from jax.experimental import pallas as pl
from jax.experimental.pallas import tpu as pltpu
from jax.experimental.pallas import tpu_sc as plsc

_B = 16384
_F = 128
_NC = 2
_NS = 16
_NW = _NC * _NS
_BPW = _B // _NW
_NCHUNK = 4
_CHUNK = _BPW // _NCHUNK
_L = 16


def _body(users_hbm, item_hbm, uemb_hbm, vemb_hbm, out_hbm,
          uidx, vidx, ubuf, vbuf, obuf, ptmp, usem, vsem):
    wid = lax.axis_index("s") * _NC + lax.axis_index("c")
    base = wid * _BPW

    for c in range(_NCHUNK):
        pltpu.sync_copy(users_hbm.at[pl.ds(base + c * _CHUNK, _CHUNK)],
                        uidx.at[c])
        pltpu.sync_copy(item_hbm.at[pl.ds(base + c * _CHUNK, _CHUNK)],
                        vidx.at[c])

    def start(c):
        b = c % 2
        cu = pltpu.make_async_copy(uemb_hbm.at[uidx.at[c]], ubuf.at[b], usem)
        cv = pltpu.make_async_copy(vemb_hbm.at[vidx.at[c]], vbuf.at[b], vsem)
        cu.start()
        cv.start()
        return cu, cv

    inflight = [start(0), start(1)]

    for c in range(_NCHUNK):
        b = c % 2
        cu, cv = inflight[c]
        cu.wait()
        cv.wait()

        rows = lax.iota(jnp.int32, _L)
        for g in range(_CHUNK // _L):
            def pair(i, _, g=g):
                p = g * _L + i
                acc = ubuf[b, p, pl.ds(0, _L)] * vbuf[b, p, pl.ds(0, _L)]
                for s in range(1, _F // _L):
                    acc += (ubuf[b, p, pl.ds(s * _L, _L)]
                            * vbuf[b, p, pl.ds(s * _L, _L)])
                ptmp[pl.ds(i * (_L + 1), _L)] = acc
                return 0

            lax.fori_loop(0, _L, pair, 0)
            flat = rows * (_L + 1)
            out_vec = plsc.load_gather(ptmp, [flat])
            for l in range(1, _L):
                out_vec += plsc.load_gather(ptmp, [flat + l])
            obuf[pl.ds(c * _CHUNK + g * _L, _L)] = out_vec

        if c + 2 < _NCHUNK:
            inflight.append(start(c + 2))

    pltpu.sync_copy(obuf, out_hbm.at[pl.ds(base, _BPW)])


@jax.jit
def _bpr_dot(users, item, user_emb, item_emb):
    mesh = plsc.VectorSubcoreMesh(core_axis_name="c", subcore_axis_name="s")
    return pl.kernel(
        _body,
        out_type=jax.ShapeDtypeStruct((_B,), jnp.float32),
        mesh=mesh,
        compiler_params=pltpu.CompilerParams(needs_layout_passes=False),
        scratch_types=[
            pltpu.VMEM((_NCHUNK, _CHUNK), jnp.int32),
            pltpu.VMEM((_NCHUNK, _CHUNK), jnp.int32),
            pltpu.VMEM((2, _CHUNK, _F), jnp.float32),
            pltpu.VMEM((2, _CHUNK, _F), jnp.float32),
            pltpu.VMEM((_BPW,), jnp.float32),
            pltpu.VMEM((_L * (_L + 1),), jnp.float32),
            pltpu.SemaphoreType.DMA,
            pltpu.SemaphoreType.DMA,
        ],
    )(users, item, user_emb, item_emb)


def kernel(users, item, user_emb, item_emb):
    return _bpr_dot(users, item, user_emb, item_emb)
